# baseline (speedup 1.0000x reference)
import jax
import jax.numpy as jnp
from jax import lax
from jax.experimental import pallas as pl
from jax.experimental.pallas import tpu as pltpu
from jax.experimental.pallas import tpu_sc as plsc

NUM_EXPERTS = 16
NBUF = 5
_SC_INFO = plsc.get_sparse_core_info()
_NC = _SC_INFO.num_cores
_TPW = 16


def _route_body(eids_hbm, wtab_hbm, mt_hbm, ids_v, wv_v, col_v):
    w = lax.axis_index("s") * _NC + lax.axis_index("c")
    n = ids_v.shape[0]

    @pl.when(w < NUM_EXPERTS)
    def _():
        pltpu.sync_copy(eids_hbm, ids_v)
        pltpu.sync_copy(wtab_hbm.at[pl.ds(w * 16, 16)], wv_v)
        wvec = wv_v[...]
        for c in range(n // 16):
            v = ids_v[pl.ds(c * 16, 16)]
            hit = 1 - jnp.minimum(jnp.abs(v - wvec), 1)
            col_v[pl.ds(c * 16, 16)] = hit.astype(jnp.float32)
        pltpu.sync_copy(col_v, mt_hbm.at[pl.ds(w * n, n)])


def _routing_mask_t(expert_ids):
    n = expert_ids.shape[0]
    wtab = jnp.repeat(jnp.arange(NUM_EXPERTS, dtype=jnp.int32), 16)
    flat = pl.kernel(
        _route_body,
        out_type=jax.ShapeDtypeStruct((NUM_EXPERTS * n,), jnp.float32),
        mesh=plsc.VectorSubcoreMesh(core_axis_name="c", subcore_axis_name="s"),
        scratch_types=[
            pltpu.VMEM((n,), jnp.int32),
            pltpu.VMEM((16,), jnp.int32),
            pltpu.VMEM((n,), jnp.float32),
        ],
    )(expert_ids, wtab)
    return flat.reshape(NUM_EXPERTS, n)


def _moe_body(mt_ref, x_ref, gw_hbm, uw_hbm, dw_hbm, out_ref,
              gbuf, ubuf, dbuf, sems):
    m = mt_ref[...].T
    def start(e):
        s = e % NBUF
        pltpu.make_async_copy(gw_hbm.at[e], gbuf.at[s], sems.at[s, 0]).start()
        pltpu.make_async_copy(uw_hbm.at[e], ubuf.at[s], sems.at[s, 1]).start()
        pltpu.make_async_copy(dw_hbm.at[e], dbuf.at[s], sems.at[s, 2]).start()

    for e in range(NBUF):
        start(e)

    x = x_ref[...]
    xb = x.astype(jnp.bfloat16)
    for e in range(NUM_EXPERTS):
        s = e % NBUF
        pltpu.make_async_copy(gw_hbm.at[e], gbuf.at[s], sems.at[s, 0]).wait()
        gate = jax.lax.dot_general(xb, gbuf[s].astype(jnp.bfloat16),
                                   (((1,), (1,)), ((), ())),
                                   preferred_element_type=jnp.float32)
        pltpu.make_async_copy(uw_hbm.at[e], ubuf.at[s], sems.at[s, 1]).wait()
        up = jax.lax.dot_general(xb, ubuf[s].astype(jnp.bfloat16),
                                 (((1,), (1,)), ((), ())),
                                 preferred_element_type=jnp.float32)
        h = gate * jax.nn.sigmoid(gate) * up
        pltpu.make_async_copy(dw_hbm.at[e], dbuf.at[s], sems.at[s, 2]).wait()
        outp = jax.lax.dot_general(h.astype(jnp.bfloat16),
                                   dbuf[s].astype(jnp.bfloat16),
                                   (((1,), (1,)), ((), ())),
                                   preferred_element_type=jnp.float32)
        contrib = outp * m[:, e:e + 1]
        if e == 0:
            out_ref[...] = contrib
        else:
            out_ref[...] += contrib
        if e + NBUF < NUM_EXPERTS:
            start(e + NBUF)


def kernel(x, expert_ids, gate_weight, up_weight, down_weight):
    n, d = x.shape
    num_e, hidden, _ = gate_weight.shape
    m = _routing_mask_t(expert_ids)
    return pl.pallas_call(
        _moe_body,
        in_specs=[
            pl.BlockSpec(memory_space=pltpu.MemorySpace.VMEM),
            pl.BlockSpec(memory_space=pltpu.MemorySpace.VMEM),
            pl.BlockSpec(memory_space=pltpu.MemorySpace.HBM),
            pl.BlockSpec(memory_space=pltpu.MemorySpace.HBM),
            pl.BlockSpec(memory_space=pltpu.MemorySpace.HBM),
        ],
        out_specs=pl.BlockSpec(memory_space=pltpu.MemorySpace.VMEM),
        out_shape=jax.ShapeDtypeStruct((n, d), jnp.float32),
        scratch_shapes=[
            pltpu.VMEM((NBUF, hidden, d), jnp.float32),
            pltpu.VMEM((NBUF, hidden, d), jnp.float32),
            pltpu.VMEM((NBUF, d, hidden), jnp.float32),
            pltpu.SemaphoreType.DMA((NBUF, 3)),
        ],
    )(m, x, gate_weight, up_weight, down_weight)


# speedup vs baseline: 1.5563x; 1.5563x over previous
import jax
import jax.numpy as jnp
from jax.experimental import pallas as pl
from jax.experimental.pallas import tpu as pltpu

NUM_EXPERTS = 16
NBUF = 5


def _moe_body(eids_ref, x_ref, gw_hbm, uw_hbm, dw_hbm, out_ref,
              gbuf, ubuf, dbuf, sems):
    def start(e):
        s = e % NBUF
        pltpu.make_async_copy(gw_hbm.at[e], gbuf.at[s], sems.at[s, 0]).start()
        pltpu.make_async_copy(uw_hbm.at[e], ubuf.at[s], sems.at[s, 1]).start()
        pltpu.make_async_copy(dw_hbm.at[e], dbuf.at[s], sems.at[s, 2]).start()

    for e in range(NBUF):
        start(e)

    x = x_ref[...]
    eids = eids_ref[...]
    for e in range(NUM_EXPERTS):
        s = e % NBUF
        xb = x.astype(jnp.bfloat16)
        pltpu.make_async_copy(gw_hbm.at[e], gbuf.at[s], sems.at[s, 0]).wait()
        gate = jax.lax.dot_general(xb, gbuf[s].astype(jnp.bfloat16),
                                   (((1,), (1,)), ((), ())),
                                   preferred_element_type=jnp.float32)
        pltpu.make_async_copy(uw_hbm.at[e], ubuf.at[s], sems.at[s, 1]).wait()
        up = jax.lax.dot_general(xb, ubuf[s].astype(jnp.bfloat16),
                                 (((1,), (1,)), ((), ())),
                                 preferred_element_type=jnp.float32)
        h = gate * jax.nn.sigmoid(gate) * up
        pltpu.make_async_copy(dw_hbm.at[e], dbuf.at[s], sems.at[s, 2]).wait()
        outp = jax.lax.dot_general(h.astype(jnp.bfloat16),
                                   dbuf[s].astype(jnp.bfloat16),
                                   (((1,), (1,)), ((), ())),
                                   preferred_element_type=jnp.float32)
        contrib = jnp.where(eids == e, outp, 0.0)
        if e == 0:
            out_ref[...] = contrib
        else:
            out_ref[...] += contrib
        if e + NBUF < NUM_EXPERTS:
            start(e + NBUF)


def kernel(x, expert_ids, gate_weight, up_weight, down_weight):
    n, d = x.shape
    num_e, hidden, _ = gate_weight.shape
    eids = expert_ids.reshape(n, 1)
    return pl.pallas_call(
        _moe_body,
        in_specs=[
            pl.BlockSpec(memory_space=pltpu.MemorySpace.VMEM),
            pl.BlockSpec(memory_space=pltpu.MemorySpace.VMEM),
            pl.BlockSpec(memory_space=pltpu.MemorySpace.HBM),
            pl.BlockSpec(memory_space=pltpu.MemorySpace.HBM),
            pl.BlockSpec(memory_space=pltpu.MemorySpace.HBM),
        ],
        out_specs=pl.BlockSpec(memory_space=pltpu.MemorySpace.VMEM),
        out_shape=jax.ShapeDtypeStruct((n, d), jnp.float32),
        scratch_shapes=[
            pltpu.VMEM((NBUF, hidden, d), jnp.float32),
            pltpu.VMEM((NBUF, hidden, d), jnp.float32),
            pltpu.VMEM((NBUF, d, hidden), jnp.float32),
            pltpu.SemaphoreType.DMA((NBUF, 3)),
        ],
    )(eids, x, gate_weight, up_weight, down_weight)
